# Initial kernel scaffold; baseline (speedup 1.0000x reference)
import jax
import jax.numpy as jnp
from jax import lax
from jax.experimental import pallas as pl
from jax.experimental.pallas import tpu as pltpu
from jax.experimental.pallas import tpu_sc as plsc

NN = 10000
NE = 320000
IN_DIM = 128
D1 = 64
NC = 2
NS = 16
LANES = 16
EPT_ALL = NE // NS
EPT_HALF = NE // (NS * NC)
CH = 80
RPT = NN // NS

_F32 = jnp.float32


def _lane_splat(v16, lane):
    idx = jnp.full((LANES,), lane, jnp.int32)
    return jnp.take(v16, idx, mode="promise_in_bounds")


def _k1_body(x_ref, w1_ref, a_ref, h_ref, s_ref):
    xb = x_ref[...]
    h = lax.dot_general(xb, w1_ref[...], (((1,), (1,)), ((), ())),
                        preferred_element_type=_F32)
    h_ref[...] = h
    s_ref[...] = lax.dot_general(a_ref[...], h, (((1,), (1,)), ((), ())),
                                 preferred_element_type=_F32)


def _proj1(x, w1, a2x64):
    blk = 1000
    return pl.pallas_call(
        _k1_body,
        grid=(NN // blk,),
        in_specs=[
            pl.BlockSpec((blk, IN_DIM), lambda i: (i, 0)),
            pl.BlockSpec((D1, IN_DIM), lambda i: (0, 0)),
            pl.BlockSpec((2, D1), lambda i: (0, 0)),
        ],
        out_specs=[
            pl.BlockSpec((blk, D1), lambda i: (i, 0)),
            pl.BlockSpec((2, blk), lambda i: (0, i)),
        ],
        out_shape=[
            jax.ShapeDtypeStruct((NN, D1), _F32),
            jax.ShapeDtypeStruct((2, NN), _F32),
        ],
    )(x, w1, a2x64)


def _gat1_body(h_hbm, s_hbm, src_hbm, dst_hbm, out_hbm, sums_hbm,
               ssrc, sdst, src1, dst1, ebuf, stage16, statsv, stage,
               src3, dst3, w3, rows, htab, accsh, wall, statssh, sem):
    c = lax.axis_index("c")
    s = lax.axis_index("s")
    gtid = c * NS + s
    row0 = s * RPT

    pltpu.sync_copy(s_hbm.at[0], ssrc)
    pltpu.sync_copy(s_hbm.at[1], sdst)

    def _zrow(r, _):
        for f in range(D1 // LANES):
            stage[r, pl.ds(f * LANES, LANES)] = jnp.zeros((LANES,), _F32)
        return 0
    lax.fori_loop(0, 125, _zrow, 0)
    for rr in range(5):
        pltpu.sync_copy(stage, accsh.at[pl.ds(row0 + rr * 125, 125)])

    for rr in range(5):
        r = row0 + rr * 125
        pltpu.sync_copy(h_hbm.at[pl.ds(r, 125)], stage)
        pltpu.sync_copy(stage, htab.at[pl.ds(r, 125)])

    e0 = s * EPT_ALL
    pltpu.sync_copy(src_hbm.at[pl.ds(e0, EPT_ALL)], src1)
    pltpu.sync_copy(dst_hbm.at[pl.ds(e0, EPT_ALL)], dst1)

    def _p1(k, m16):
        i16 = src1[pl.ds(k * LANES, LANES)]
        j16 = dst1[pl.ds(k * LANES, LANES)]
        u = plsc.load_gather(ssrc, [i16]) + plsc.load_gather(sdst, [j16])
        e16 = jnp.where(u >= 0.0, u, 0.01 * u)
        ebuf[pl.ds(k * LANES, LANES)] = e16
        return jnp.maximum(m16, e16)

    m16 = lax.fori_loop(0, EPT_ALL // LANES, _p1,
                        jnp.full((LANES,), -jnp.inf, _F32))
    stage16[...] = m16
    pltpu.sync_copy(stage16, statssh.at[pl.ds(s * LANES, LANES)])
    plsc.subcore_barrier()

    pltpu.sync_copy(statssh, statsv)

    def _rmax(i, mm):
        return jnp.maximum(mm, statsv[pl.ds(i * LANES, LANES)])

    m16 = lax.fori_loop(0, NS, _rmax, jnp.full((LANES,), -jnp.inf, _F32))
    m = jnp.max(m16)

    def _p2(k, s16):
        w16 = jnp.exp(ebuf[pl.ds(k * LANES, LANES)] - m)
        ebuf[pl.ds(k * LANES, LANES)] = w16
        return s16 + w16

    sum16 = lax.fori_loop(0, EPT_ALL // LANES, _p2, jnp.zeros((LANES,), _F32))
    stage16[...] = sum16
    pltpu.sync_copy(stage16, sums_hbm.at[gtid])
    pltpu.sync_copy(ebuf, wall.at[pl.ds(e0, EPT_ALL)])
    plsc.subcore_barrier()

    base0 = gtid * EPT_HALF

    def _p3(q, _):
        b = base0 + q * CH
        pltpu.sync_copy(src_hbm.at[pl.ds(b, CH)], src3)
        pltpu.sync_copy(dst_hbm.at[pl.ds(b, CH)], dst3)
        pltpu.sync_copy(wall.at[pl.ds(b, CH)], w3)
        pltpu.async_copy(htab.at[src3], rows, sem).wait()
        for g in range(CH // LANES):
            w16 = w3[pl.ds(g * LANES, LANES)]
            for l in range(LANES):
                e = g * LANES + l
                wsp = _lane_splat(w16, l)
                for f in range(D1 // LANES):
                    sl = pl.ds(f * LANES, LANES)
                    rows[e, sl] = rows[e, sl] * wsp
        pltpu.sync_copy(rows, accsh.at[dst3], add=True)
        return 0

    lax.fori_loop(0, EPT_HALF // CH, _p3, 0)
    plsc.subcore_barrier()

    for rr in range(5):
        r = row0 + rr * 125
        pltpu.sync_copy(accsh.at[pl.ds(r, 125)], stage)
        pltpu.sync_copy(stage, out_hbm.at[c, pl.ds(r, 125)])


def _gat1_edges(h1, s1, src, dst):
    mesh = plsc.VectorSubcoreMesh(core_axis_name="c", subcore_axis_name="s",
                                  num_cores=NC, num_subcores=NS)
    f = pl.kernel(
        _gat1_body,
        out_type=[
            jax.ShapeDtypeStruct((NC, NN, D1), _F32),
            jax.ShapeDtypeStruct((NC * NS, LANES), _F32),
        ],
        mesh=mesh,
        scratch_types=[
            pltpu.VMEM((NN,), _F32),
            pltpu.VMEM((NN,), _F32),
            pltpu.VMEM((EPT_ALL,), jnp.int32),
            pltpu.VMEM((EPT_ALL,), jnp.int32),
            pltpu.VMEM((EPT_ALL,), _F32),
            pltpu.VMEM((LANES,), _F32),
            pltpu.VMEM((NS * LANES,), _F32),
            pltpu.VMEM((125, D1), _F32),
            pltpu.VMEM((CH,), jnp.int32),
            pltpu.VMEM((CH,), jnp.int32),
            pltpu.VMEM((CH,), _F32),
            pltpu.VMEM((CH, D1), _F32),
            pltpu.VMEM_SHARED((NN, D1), _F32),
            pltpu.VMEM_SHARED((NN, D1), _F32),
            pltpu.VMEM_SHARED((NE,), _F32),
            pltpu.VMEM_SHARED((NS * LANES,), _F32),
            pltpu.SemaphoreType.DMA,
        ],
    )
    return f(h1, s1, src, dst)


def _k3_body(parts_ref, sums_ref, w2_ref, a2_ref, h2_ref, s2_ref):
    z1 = jnp.sum(sums_ref[...]) * 0.5
    p = parts_ref[0] + parts_ref[1]
    x1 = p * (1.0 / z1)
    x1 = jnp.where(x1 > 0.0, x1, jnp.expm1(x1))
    h2t = lax.dot_general(w2_ref[...], x1, (((1,), (1,)), ((), ())),
                          preferred_element_type=_F32)
    h2_ref[...] = h2t
    a_src = a2_ref[0, 0]
    a_dst = a2_ref[0, 1]
    s2_ref[...] = jnp.concatenate([a_src * h2t, a_dst * h2t], axis=0)


def _mid(parts, sums1, w2, a2):
    blk = 1000
    return pl.pallas_call(
        _k3_body,
        grid=(NN // blk,),
        in_specs=[
            pl.BlockSpec((NC, blk, D1), lambda i: (0, i, 0)),
            pl.BlockSpec((NC * NS, LANES), lambda i: (0, 0)),
            pl.BlockSpec((1, D1), lambda i: (0, 0)),
            pl.BlockSpec((1, 2), lambda i: (0, 0)),
        ],
        out_specs=[
            pl.BlockSpec((1, blk), lambda i: (0, i)),
            pl.BlockSpec((2, blk), lambda i: (0, i)),
        ],
        out_shape=[
            jax.ShapeDtypeStruct((1, NN), _F32),
            jax.ShapeDtypeStruct((2, NN), _F32),
        ],
    )(parts, sums1, w2, a2)


def _gat2_body(h_hbm, s_hbm, src_hbm, dst_hbm, out_hbm, sums_hbm,
               ssrc, sdst, h2tab, src1, dst1, ebuf, stage16, statsv,
               zstage, src3, dst3, w3, msg, acc2sh, wall, statssh):
    c = lax.axis_index("c")
    s = lax.axis_index("s")
    gtid = c * NS + s

    pltpu.sync_copy(s_hbm.at[0], ssrc)
    pltpu.sync_copy(s_hbm.at[1], sdst)
    pltpu.sync_copy(h_hbm, h2tab)

    def _z(i, _):
        zstage[pl.ds(i * LANES, LANES)] = jnp.zeros((LANES,), _F32)
        return 0
    lax.fori_loop(0, 40, _z, 0)

    @pl.when(s < NS - 1)
    def _():
        pltpu.sync_copy(zstage.at[pl.ds(0, 640)],
                        acc2sh.at[pl.ds(s * 640, 640)])

    @pl.when(s == NS - 1)
    def _():
        pltpu.sync_copy(zstage.at[pl.ds(0, 400)], acc2sh.at[pl.ds(9600, 400)])

    e0 = s * EPT_ALL
    pltpu.sync_copy(src_hbm.at[pl.ds(e0, EPT_ALL)], src1)
    pltpu.sync_copy(dst_hbm.at[pl.ds(e0, EPT_ALL)], dst1)

    def _p1(k, m16):
        i16 = src1[pl.ds(k * LANES, LANES)]
        j16 = dst1[pl.ds(k * LANES, LANES)]
        u = plsc.load_gather(ssrc, [i16]) + plsc.load_gather(sdst, [j16])
        e16 = jnp.where(u >= 0.0, u, 0.01 * u)
        ebuf[pl.ds(k * LANES, LANES)] = e16
        return jnp.maximum(m16, e16)

    m16 = lax.fori_loop(0, EPT_ALL // LANES, _p1,
                        jnp.full((LANES,), -jnp.inf, _F32))
    stage16[...] = m16
    pltpu.sync_copy(stage16, statssh.at[pl.ds(s * LANES, LANES)])
    plsc.subcore_barrier()

    pltpu.sync_copy(statssh, statsv)

    def _rmax(i, mm):
        return jnp.maximum(mm, statsv[pl.ds(i * LANES, LANES)])

    m16 = lax.fori_loop(0, NS, _rmax, jnp.full((LANES,), -jnp.inf, _F32))
    m = jnp.max(m16)

    def _p2(k, s16):
        w16 = jnp.exp(ebuf[pl.ds(k * LANES, LANES)] - m)
        ebuf[pl.ds(k * LANES, LANES)] = w16
        return s16 + w16

    sum16 = lax.fori_loop(0, EPT_ALL // LANES, _p2, jnp.zeros((LANES,), _F32))
    stage16[...] = sum16
    pltpu.sync_copy(stage16, sums_hbm.at[gtid])
    pltpu.sync_copy(ebuf, wall.at[pl.ds(e0, EPT_ALL)])
    plsc.subcore_barrier()

    base0 = gtid * EPT_HALF

    def _p3(q, _):
        b = base0 + q * CH
        pltpu.sync_copy(src_hbm.at[pl.ds(b, CH)], src3)
        pltpu.sync_copy(dst_hbm.at[pl.ds(b, CH)], dst3)
        pltpu.sync_copy(wall.at[pl.ds(b, CH)], w3)
        for g in range(CH // LANES):
            sl = pl.ds(g * LANES, LANES)
            i16 = src3[sl]
            msg[sl] = plsc.load_gather(h2tab, [i16]) * w3[sl]
        pltpu.sync_copy(msg, acc2sh.at[dst3], add=True)
        return 0

    lax.fori_loop(0, EPT_HALF // CH, _p3, 0)
    plsc.subcore_barrier()

    @pl.when(s < NS - 1)
    def _():
        pltpu.sync_copy(acc2sh.at[pl.ds(s * 640, 640)],
                        zstage.at[pl.ds(0, 640)])
        pltpu.sync_copy(zstage.at[pl.ds(0, 640)],
                        out_hbm.at[c, pl.ds(s * 640, 640)])

    @pl.when(s == NS - 1)
    def _():
        pltpu.sync_copy(acc2sh.at[pl.ds(9600, 400)], zstage.at[pl.ds(0, 400)])
        pltpu.sync_copy(zstage.at[pl.ds(0, 400)],
                        out_hbm.at[c, pl.ds(9600, 400)])


def _gat2_edges(h2, s2, src, dst):
    mesh = plsc.VectorSubcoreMesh(core_axis_name="c", subcore_axis_name="s",
                                  num_cores=NC, num_subcores=NS)
    f = pl.kernel(
        _gat2_body,
        out_type=[
            jax.ShapeDtypeStruct((NC, NN), _F32),
            jax.ShapeDtypeStruct((NC * NS, LANES), _F32),
        ],
        mesh=mesh,
        scratch_types=[
            pltpu.VMEM((NN,), _F32),
            pltpu.VMEM((NN,), _F32),
            pltpu.VMEM((NN,), _F32),
            pltpu.VMEM((EPT_ALL,), jnp.int32),
            pltpu.VMEM((EPT_ALL,), jnp.int32),
            pltpu.VMEM((EPT_ALL,), _F32),
            pltpu.VMEM((LANES,), _F32),
            pltpu.VMEM((NS * LANES,), _F32),
            pltpu.VMEM((640,), _F32),
            pltpu.VMEM((CH,), jnp.int32),
            pltpu.VMEM((CH,), jnp.int32),
            pltpu.VMEM((CH,), _F32),
            pltpu.VMEM((CH,), _F32),
            pltpu.VMEM_SHARED((NN,), _F32),
            pltpu.VMEM_SHARED((NE,), _F32),
            pltpu.VMEM_SHARED((NS * LANES,), _F32),
        ],
    )
    return f(h2, s2, src, dst)


def _k5_body(parts_ref, sums_ref, ci_ref, cr_ref, n_ref, i_ref, r_ref,
             wih_ref, bih_ref, bhh_ref,
             wih_h_ref, wic_ref, bi_ref, wrh_ref, wrc_ref, br_ref,
             wsh_ref, wsc_ref, bs_ref,
             predI_ref, predR_ref, phyI_ref, phyR_ref, h_ref):
    z2 = jnp.sum(sums_ref[...]) * 0.5
    p = parts_ref[0:1, :] + parts_ref[1:2, :]
    x2 = p * (1.0 / z2)
    x2 = jnp.where(x2 > 0.0, x2, jnp.expm1(x2))

    gi = wih_ref[...] * x2 + bih_ref[...]
    gh = bhh_ref[...]
    G = 32
    r = jax.nn.sigmoid(gi[0:G, :] + gh[0:G, :])
    z = jax.nn.sigmoid(gi[G:2 * G, :] + gh[G:2 * G, :])
    n = jnp.tanh(gi[2 * G:3 * G, :] + r * gh[2 * G:3 * G, :])
    h_new = (1.0 - z) * n
    h_ref[...] = h_new

    ci = ci_ref[...]
    cr = cr_ref[...]

    def head(wh_ref, wc_ref, b_ref):
        t = lax.dot_general(wh_ref[...], h_new, (((1,), (0,)), ((), ())),
                            preferred_element_type=_F32)
        return t + wc_ref[:, 0:1] * ci + wc_ref[:, 1:2] * cr + b_ref[...]

    predI_ref[...] = head(wih_h_ref, wic_ref, bi_ref)
    predR_ref[...] = head(wrh_ref, wrc_ref, br_ref)
    sir = jax.nn.sigmoid(head(wsh_ref, wsc_ref, bs_ref))
    alpha = sir[0:1, :]
    beta = sir[1:2, :]
    Nv = n_ref[...]
    Iv = i_ref[...]
    Rv = r_ref[...]
    Sv = jnp.maximum(Nv - Iv - Rv, 0.0)
    phyI_ref[...] = alpha * Iv * (Sv / Nv) - beta * Iv
    phyR_ref[...] = beta * Iv


def _final(parts2, sums2, ci, cr, nv, iv, rv, wih, bih, bhh,
           wi_h, wi_c, bi, wr_h, wr_c, br, ws_h, ws_c, bs):
    blk = 1000
    PW = 5
    G = 32
    c0 = lambda i: (0, 0)
    vec = pl.BlockSpec((1, blk), lambda i: (0, i))
    return pl.pallas_call(
        _k5_body,
        grid=(NN // blk,),
        in_specs=[
            pl.BlockSpec((NC, blk), lambda i: (0, i)),
            pl.BlockSpec((NC * NS, LANES), c0),
            vec, vec, vec, vec, vec,
            pl.BlockSpec((3 * G, 1), c0),
            pl.BlockSpec((3 * G, 1), c0),
            pl.BlockSpec((3 * G, 1), c0),
            pl.BlockSpec((PW, G), c0),
            pl.BlockSpec((PW, 2), c0),
            pl.BlockSpec((PW, 1), c0),
            pl.BlockSpec((PW, G), c0),
            pl.BlockSpec((PW, 2), c0),
            pl.BlockSpec((PW, 1), c0),
            pl.BlockSpec((2, G), c0),
            pl.BlockSpec((2, 2), c0),
            pl.BlockSpec((2, 1), c0),
        ],
        out_specs=[
            pl.BlockSpec((PW, blk), lambda i: (0, i)),
            pl.BlockSpec((PW, blk), lambda i: (0, i)),
            pl.BlockSpec((1, blk), lambda i: (0, i)),
            pl.BlockSpec((1, blk), lambda i: (0, i)),
            pl.BlockSpec((G, blk), lambda i: (0, i)),
        ],
        out_shape=[
            jax.ShapeDtypeStruct((PW, NN), _F32),
            jax.ShapeDtypeStruct((PW, NN), _F32),
            jax.ShapeDtypeStruct((1, NN), _F32),
            jax.ShapeDtypeStruct((1, NN), _F32),
            jax.ShapeDtypeStruct((G, NN), _F32),
        ],
    )(parts2, sums2, ci, cr, nv, iv, rv, wih, bih, bhh,
      wi_h, wi_c, bi, wr_h, wr_c, br, ws_h, ws_c, bs)


def kernel(dynamic, cI, cR, N, I, R, edge_index, W1, a1, W2, a2,
           W_ih, W_hh, b_ih, b_hh, WI, bI, WR, bR, Ws, bs):
    x = dynamic.reshape(NN, IN_DIM)
    src = edge_index[0]
    dst = edge_index[1]

    h1, s1 = _proj1(x, W1, a1.reshape(2, D1))
    parts1, sums1 = _gat1_edges(h1, s1, src, dst)

    h2row, s2 = _mid(parts1, sums1, W2, a2)
    parts2, sums2 = _gat2_edges(h2row.reshape(NN), s2, src, dst)

    G = 32
    PW = 5
    pI, pR, fI, fR, hT = _final(
        parts2, sums2, cI, cR,
        N.reshape(1, NN), I.reshape(1, NN), R.reshape(1, NN),
        W_ih, b_ih.reshape(3 * G, 1), b_hh.reshape(3 * G, 1),
        WI[:, :G], WI[:, G:], bI.reshape(PW, 1),
        WR[:, :G], WR[:, G:], bR.reshape(PW, 1),
        Ws[:, :G], Ws[:, G:], bs.reshape(2, 1),
    )

    pred_I = pI.T.reshape(NN, 1, PW)
    pred_R = pR.T.reshape(NN, 1, PW)
    phy_I = fI.T.reshape(NN, 1, 1)
    phy_R = fR.T.reshape(NN, 1, 1)
    h_state = hT.T.reshape(NN, 1, G)
    return (pred_I, pred_R, phy_I, phy_R, h_state)


# speedup vs baseline: 22.2477x; 22.2477x over previous
import jax
import jax.numpy as jnp
from jax import lax
from jax.experimental import pallas as pl
from jax.experimental.pallas import tpu as pltpu
from jax.experimental.pallas import tpu_sc as plsc

NN = 10000
NE = 320000
IN_DIM = 128
D1 = 64
NC = 2
NS = 16
LANES = 16
EPT = NE // (NS * NC)
CH = 80
NCHUNK = EPT // CH

_F32 = jnp.float32


def _splat_from(ref, pos):
    idx = jnp.full((LANES,), pos, jnp.int32)
    return plsc.load_gather(ref, [idx])


def _k1_body(x_ref, w1_ref, a_ref, h_ref, s_ref, m_ref):
    xb = x_ref[...]
    h = lax.dot_general(xb, w1_ref[...], (((1,), (1,)), ((), ())),
                        preferred_element_type=_F32)
    h_ref[...] = h
    s = lax.dot_general(a_ref[...], h, (((1,), (1,)), ((), ())),
                        preferred_element_type=_F32)
    s_ref[...] = s
    b = jnp.max(s[0]) + jnp.max(s[1])
    mb = jnp.where(b >= 0.0, b, 0.01 * b)
    m_ref[...] = jnp.broadcast_to(mb, (1, LANES))


def _proj1(x, w1, a2x64):
    return pl.pallas_call(
        _k1_body,
        out_shape=[
            jax.ShapeDtypeStruct((NN, D1), _F32),
            jax.ShapeDtypeStruct((2, NN), _F32),
            jax.ShapeDtypeStruct((1, LANES), _F32),
        ],
    )(x, w1, a2x64)


def _gat1_body(h_hbm, ssrc_hbm, sdst_hbm, src_hbm, dst_hbm, m_hbm,
               out_hbm, sums_hbm,
               ssrc, sdst, mbuf, srcL, dstL, w3, stage16, rows, accsh, sem):
    c = lax.axis_index("c")
    s = lax.axis_index("s")
    gtid = c * NS + s

    pltpu.sync_copy(ssrc_hbm, ssrc)
    pltpu.sync_copy(sdst_hbm, sdst)
    pltpu.sync_copy(m_hbm, mbuf)
    e0 = gtid * EPT
    pltpu.sync_copy(src_hbm.at[pl.ds(e0, EPT)], srcL)
    pltpu.sync_copy(dst_hbm.at[pl.ds(e0, EPT)], dstL)
    mv = mbuf[0, pl.ds(0, LANES)]

    def _zrow(i, _):
        for f in range(D1 // LANES):
            rows[i, pl.ds(f * LANES, LANES)] = jnp.zeros((LANES,), _F32)
        return 0
    lax.fori_loop(0, CH, _zrow, 0)
    for i in range(8):
        cid = s + i * NS
        @pl.when(cid < NN // CH)
        def _():
            pltpu.sync_copy(rows, accsh.at[pl.ds(cid * CH, CH)])
    plsc.subcore_barrier()

    def _p(q, s16):
        b = q * CH
        for g in range(CH // LANES):
            sl = pl.ds(b + g * LANES, LANES)
            u = plsc.load_gather(ssrc, [srcL[sl]]) \
                + plsc.load_gather(sdst, [dstL[sl]])
            e16 = jnp.where(u >= 0.0, u, 0.01 * u)
            w16 = jnp.exp(e16 - mv)
            w3[pl.ds(g * LANES, LANES)] = w16
            s16 = s16 + w16
        pltpu.async_copy(h_hbm.at[srcL.at[pl.ds(b, CH)]], rows, sem).wait()
        for e in range(CH):
            wsp = _splat_from(w3, e)
            for f in range(D1 // LANES):
                sl = pl.ds(f * LANES, LANES)
                rows[e, sl] = rows[e, sl] * wsp
        pltpu.sync_copy(rows, accsh.at[dstL.at[pl.ds(b, CH)]], add=True)
        return s16

    sum16 = lax.fori_loop(0, NCHUNK, _p, jnp.zeros((LANES,), _F32))
    stage16[...] = sum16
    pltpu.sync_copy(stage16, sums_hbm.at[pl.ds(gtid * LANES, LANES)])
    plsc.subcore_barrier()

    for i in range(8):
        cid = s + i * NS
        @pl.when(cid < NN // CH)
        def _():
            pltpu.sync_copy(accsh.at[pl.ds(cid * CH, CH)], rows)
            pltpu.sync_copy(rows, out_hbm.at[c, pl.ds(cid * CH, CH)])


def _gat1_edges(h1, ssrc_arr, sdst_arr, src, dst, m1):
    mesh = plsc.VectorSubcoreMesh(core_axis_name="c", subcore_axis_name="s",
                                  num_cores=NC, num_subcores=NS)
    f = pl.kernel(
        _gat1_body,
        out_type=[
            jax.ShapeDtypeStruct((NC, NN, D1), _F32),
            jax.ShapeDtypeStruct((NC * NS * LANES,), _F32),
        ],
        mesh=mesh,
        compiler_params=pltpu.CompilerParams(needs_layout_passes=False,
                                             use_tc_tiling_on_sc=False),
        scratch_types=[
            pltpu.VMEM((NN,), _F32),
            pltpu.VMEM((NN,), _F32),
            pltpu.VMEM((1, LANES), _F32),
            pltpu.VMEM((EPT,), jnp.int32),
            pltpu.VMEM((EPT,), jnp.int32),
            pltpu.VMEM((CH,), _F32),
            pltpu.VMEM((LANES,), _F32),
            pltpu.VMEM((CH, D1), _F32),
            pltpu.VMEM_SHARED((NN, D1), _F32),
            pltpu.SemaphoreType.DMA,
        ],
    )
    return f(h1, ssrc_arr, sdst_arr, src, dst, m1)


def _k3_body(parts_ref, sums_ref, w2_ref, a2_ref, h2_ref, s2_ref, m_ref):
    z1 = jnp.sum(sums_ref[...])
    v = parts_ref[...]
    p = v[0] + v[1]
    x1 = p * (1.0 / z1)
    x1 = jnp.where(x1 > 0.0, x1, jnp.exp(jnp.minimum(x1, 0.0)) - 1.0)
    h2t = lax.dot_general(w2_ref[...], x1, (((1,), (1,)), ((), ())),
                          preferred_element_type=_F32)
    h2_ref[...] = h2t
    a_src = a2_ref[0, 0]
    a_dst = a2_ref[0, 1]
    s2 = jnp.concatenate([a_src * h2t, a_dst * h2t], axis=0)
    s2_ref[...] = s2
    b = jnp.max(s2[0]) + jnp.max(s2[1])
    mb = jnp.where(b >= 0.0, b, 0.01 * b)
    m_ref[...] = jnp.broadcast_to(mb, (1, LANES))


def _mid(parts, sums1, w2, a2):
    return pl.pallas_call(
        _k3_body,
        out_shape=[
            jax.ShapeDtypeStruct((1, NN), _F32),
            jax.ShapeDtypeStruct((2, NN), _F32),
            jax.ShapeDtypeStruct((1, LANES), _F32),
        ],
    )(parts, sums1, w2, a2)


def _gat2_body(h_hbm, ssrc_hbm, sdst_hbm, src_hbm, dst_hbm, m_hbm,
               out_hbm, sums_hbm,
               ssrc, sdst, h2tab, mbuf, srcL, dstL, msg, stage16, zstage,
               acc2sh):
    c = lax.axis_index("c")
    s = lax.axis_index("s")
    gtid = c * NS + s

    pltpu.sync_copy(ssrc_hbm, ssrc)
    pltpu.sync_copy(sdst_hbm, sdst)
    pltpu.sync_copy(h_hbm, h2tab)
    pltpu.sync_copy(m_hbm, mbuf)
    e0 = gtid * EPT
    pltpu.sync_copy(src_hbm.at[pl.ds(e0, EPT)], srcL)
    pltpu.sync_copy(dst_hbm.at[pl.ds(e0, EPT)], dstL)
    mv = mbuf[0, pl.ds(0, LANES)]

    def _z(i, _):
        zstage[pl.ds(i * LANES, LANES)] = jnp.zeros((LANES,), _F32)
        return 0
    lax.fori_loop(0, 40, _z, 0)

    @pl.when(s < NS - 1)
    def _():
        pltpu.sync_copy(zstage.at[pl.ds(0, 640)],
                        acc2sh.at[pl.ds(s * 640, 640)])

    @pl.when(s == NS - 1)
    def _():
        pltpu.sync_copy(zstage.at[pl.ds(0, 400)], acc2sh.at[pl.ds(9600, 400)])
    plsc.subcore_barrier()

    def _p(q, s16):
        b = q * CH
        for g in range(CH // LANES):
            sl = pl.ds(b + g * LANES, LANES)
            i16 = srcL[sl]
            u = plsc.load_gather(ssrc, [i16]) \
                + plsc.load_gather(sdst, [dstL[sl]])
            e16 = jnp.where(u >= 0.0, u, 0.01 * u)
            w16 = jnp.exp(e16 - mv)
            msg[pl.ds(g * LANES, LANES)] = plsc.load_gather(h2tab, [i16]) * w16
            s16 = s16 + w16
        pltpu.sync_copy(msg, acc2sh.at[dstL.at[pl.ds(b, CH)]], add=True)
        return s16

    sum16 = lax.fori_loop(0, NCHUNK, _p, jnp.zeros((LANES,), _F32))
    stage16[...] = sum16
    pltpu.sync_copy(stage16, sums_hbm.at[pl.ds(gtid * LANES, LANES)])
    plsc.subcore_barrier()

    @pl.when(s < NS - 1)
    def _():
        pltpu.sync_copy(acc2sh.at[pl.ds(s * 640, 640)],
                        zstage.at[pl.ds(0, 640)])
        pltpu.sync_copy(zstage.at[pl.ds(0, 640)],
                        out_hbm.at[pl.ds(c * NN + s * 640, 640)])

    @pl.when(s == NS - 1)
    def _():
        pltpu.sync_copy(acc2sh.at[pl.ds(9600, 400)], zstage.at[pl.ds(0, 400)])
        pltpu.sync_copy(zstage.at[pl.ds(0, 400)],
                        out_hbm.at[pl.ds(c * NN + 9600, 400)])


def _gat2_edges(h2, ssrc_arr, sdst_arr, src, dst, m2):
    mesh = plsc.VectorSubcoreMesh(core_axis_name="c", subcore_axis_name="s",
                                  num_cores=NC, num_subcores=NS)
    f = pl.kernel(
        _gat2_body,
        out_type=[
            jax.ShapeDtypeStruct((NC * NN,), _F32),
            jax.ShapeDtypeStruct((NC * NS * LANES,), _F32),
        ],
        mesh=mesh,
        compiler_params=pltpu.CompilerParams(needs_layout_passes=False,
                                             use_tc_tiling_on_sc=False),
        scratch_types=[
            pltpu.VMEM((NN,), _F32),
            pltpu.VMEM((NN,), _F32),
            pltpu.VMEM((NN,), _F32),
            pltpu.VMEM((1, LANES), _F32),
            pltpu.VMEM((EPT,), jnp.int32),
            pltpu.VMEM((EPT,), jnp.int32),
            pltpu.VMEM((CH,), _F32),
            pltpu.VMEM((LANES,), _F32),
            pltpu.VMEM((640,), _F32),
            pltpu.VMEM_SHARED((NN,), _F32),
        ],
    )
    return f(h2, ssrc_arr, sdst_arr, src, dst, m2)


def _k5_body(parts_ref, sums_ref, ci_ref, cr_ref, n_ref, i_ref, r_ref,
             wih_ref, bih_ref, bhh_ref,
             wih_h_ref, wic_ref, bi_ref, wrh_ref, wrc_ref, br_ref,
             wsh_ref, wsc_ref, bs_ref,
             predI_ref, predR_ref, phyI_ref, phyR_ref, h_ref):
    z2 = jnp.sum(sums_ref[...])
    v = parts_ref[...]
    p = v[0:1, :] + v[1:2, :]
    x2 = p * (1.0 / z2)
    x2 = jnp.where(x2 > 0.0, x2, jnp.exp(jnp.minimum(x2, 0.0)) - 1.0)

    gi = wih_ref[...] * x2 + bih_ref[...]
    gh = bhh_ref[...]
    G = 32
    r = jax.nn.sigmoid(gi[0:G, :] + gh[0:G, :])
    z = jax.nn.sigmoid(gi[G:2 * G, :] + gh[G:2 * G, :])
    n = jnp.tanh(gi[2 * G:3 * G, :] + r * gh[2 * G:3 * G, :])
    h_new = (1.0 - z) * n
    h_ref[...] = h_new

    ci = ci_ref[...]
    cr = cr_ref[...]

    def head(wh_ref, wc_ref, b_ref):
        t = lax.dot_general(wh_ref[...], h_new, (((1,), (0,)), ((), ())),
                            preferred_element_type=_F32)
        return t + wc_ref[:, 0:1] * ci + wc_ref[:, 1:2] * cr + b_ref[...]

    predI_ref[...] = head(wih_h_ref, wic_ref, bi_ref)
    predR_ref[...] = head(wrh_ref, wrc_ref, br_ref)
    sir = jax.nn.sigmoid(head(wsh_ref, wsc_ref, bs_ref))
    alpha = sir[0:1, :]
    beta = sir[1:2, :]
    Nv = n_ref[...]
    Iv = i_ref[...]
    Rv = r_ref[...]
    Sv = jnp.maximum(Nv - Iv - Rv, 0.0)
    phyI_ref[...] = alpha * Iv * (Sv / Nv) - beta * Iv
    phyR_ref[...] = beta * Iv


def _final(parts2, sums2, ci, cr, nv, iv, rv, wih, bih, bhh,
           wi_h, wi_c, bi, wr_h, wr_c, br, ws_h, ws_c, bs):
    PW = 5
    G = 32
    return pl.pallas_call(
        _k5_body,
        out_shape=[
            jax.ShapeDtypeStruct((PW, NN), _F32),
            jax.ShapeDtypeStruct((PW, NN), _F32),
            jax.ShapeDtypeStruct((1, NN), _F32),
            jax.ShapeDtypeStruct((1, NN), _F32),
            jax.ShapeDtypeStruct((G, NN), _F32),
        ],
    )(parts2, sums2, ci, cr, nv, iv, rv, wih, bih, bhh,
      wi_h, wi_c, bi, wr_h, wr_c, br, ws_h, ws_c, bs)


def kernel(dynamic, cI, cR, N, I, R, edge_index, W1, a1, W2, a2,
           W_ih, W_hh, b_ih, b_hh, WI, bI, WR, bR, Ws, bs):
    x = dynamic.reshape(NN, IN_DIM)
    src = edge_index[0]
    dst = edge_index[1]

    h1, s1, m1 = _proj1(x, W1, a1.reshape(2, D1))
    parts1, sums1 = _gat1_edges(h1, s1[0], s1[1], src, dst, m1)

    h2row, s2, m2 = _mid(parts1, sums1, W2, a2)
    parts2, sums2 = _gat2_edges(h2row.reshape(NN), s2[0], s2[1], src, dst, m2)

    G = 32
    PW = 5
    pI, pR, fI, fR, hT = _final(
        parts2.reshape(NC, NN), sums2, cI, cR,
        N.reshape(1, NN), I.reshape(1, NN), R.reshape(1, NN),
        W_ih, b_ih.reshape(3 * G, 1), b_hh.reshape(3 * G, 1),
        WI[:, :G], WI[:, G:], bI.reshape(PW, 1),
        WR[:, :G], WR[:, G:], bR.reshape(PW, 1),
        Ws[:, :G], Ws[:, G:], bs.reshape(2, 1),
    )

    pred_I = pI.T.reshape(NN, 1, PW)
    pred_R = pR.T.reshape(NN, 1, PW)
    phy_I = fI.T.reshape(NN, 1, 1)
    phy_R = fR.T.reshape(NN, 1, 1)
    h_state = hT.T.reshape(NN, 1, G)
    return (pred_I, pred_R, phy_I, phy_R, h_state)


# speedup vs baseline: 25.1145x; 1.1289x over previous
import jax
import jax.numpy as jnp
from jax import lax
from jax.experimental import pallas as pl
from jax.experimental.pallas import tpu as pltpu
from jax.experimental.pallas import tpu_sc as plsc

NN = 10000
NE = 320000
IN_DIM = 128
D1 = 64
NC = 2
NS = 16
LANES = 16
EPT = NE // (NS * NC)
CH = 80
NCHUNK = EPT // CH

_F32 = jnp.float32


def _splat_from(ref, pos):
    idx = jnp.full((LANES,), pos, jnp.int32)
    return plsc.load_gather(ref, [idx])


def _k1_body(x_ref, w1_ref, a_ref, h_ref, s_ref, m_ref):
    xb = x_ref[...]
    h = lax.dot_general(xb, w1_ref[...], (((1,), (1,)), ((), ())),
                        preferred_element_type=_F32)
    h_ref[...] = h
    s = lax.dot_general(a_ref[...], h, (((1,), (1,)), ((), ())),
                        preferred_element_type=_F32)
    s_ref[...] = s
    b = jnp.max(s[0]) + jnp.max(s[1])
    mb = jnp.where(b >= 0.0, b, 0.01 * b)
    m_ref[...] = jnp.broadcast_to(mb, (1, LANES))


def _proj1(x, w1, a2x64):
    return pl.pallas_call(
        _k1_body,
        out_shape=[
            jax.ShapeDtypeStruct((NN, D1), _F32),
            jax.ShapeDtypeStruct((2, NN), _F32),
            jax.ShapeDtypeStruct((1, LANES), _F32),
        ],
    )(x, w1, a2x64)


def _gat1_body(h_hbm, ssrc_hbm, sdst_hbm, src_hbm, dst_hbm, m_hbm,
               out_hbm, sums_hbm,
               ssrc, sdst, mbuf, srcL, dstL, w3, stage16, rows, rowsB,
               accsh, sem, semB):
    c = lax.axis_index("c")
    s = lax.axis_index("s")
    gtid = c * NS + s

    pltpu.sync_copy(ssrc_hbm, ssrc)
    pltpu.sync_copy(sdst_hbm, sdst)
    pltpu.sync_copy(m_hbm, mbuf)
    e0 = gtid * EPT
    pltpu.sync_copy(src_hbm.at[pl.ds(e0, EPT)], srcL)
    pltpu.sync_copy(dst_hbm.at[pl.ds(e0, EPT)], dstL)
    mv = mbuf[0, pl.ds(0, LANES)]

    def _zrow(i, _):
        for f in range(D1 // LANES):
            rows[i, pl.ds(f * LANES, LANES)] = jnp.zeros((LANES,), _F32)
        return 0
    lax.fori_loop(0, CH, _zrow, 0)
    for i in range(8):
        cid = s + i * NS
        @pl.when(cid < NN // CH)
        def _():
            pltpu.sync_copy(rows, accsh.at[pl.ds(cid * CH, CH)])
    plsc.subcore_barrier()

    def _issue(q, buf, psem):
        return pltpu.async_copy(h_hbm.at[srcL.at[pl.ds(q * CH, CH)]],
                                buf, psem)

    def _consume(q, buf, psem, s16):
        b = q * CH
        pltpu.make_async_copy(h_hbm.at[srcL.at[pl.ds(b, CH)]],
                              buf, psem).wait()
        for g in range(CH // LANES):
            sl = pl.ds(b + g * LANES, LANES)
            u = plsc.load_gather(ssrc, [srcL[sl]]) \
                + plsc.load_gather(sdst, [dstL[sl]])
            e16 = jnp.where(u >= 0.0, u, 0.01 * u)
            w16 = jnp.exp(e16 - mv)
            w3[pl.ds(g * LANES, LANES)] = w16
            s16 = s16 + w16
        for e in range(CH):
            wsp = _splat_from(w3, e)
            for f in range(D1 // LANES):
                sl = pl.ds(f * LANES, LANES)
                buf[e, sl] = buf[e, sl] * wsp
        pltpu.sync_copy(buf, accsh.at[dstL.at[pl.ds(b, CH)]], add=True)
        return s16

    _issue(0, rows, sem)

    def _p(q2, s16):
        a = 2 * q2
        _issue(a + 1, rowsB, semB)
        s16 = _consume(a, rows, sem, s16)
        _issue(a + 2, rows, sem)
        s16 = _consume(a + 1, rowsB, semB, s16)
        return s16

    sum16 = lax.fori_loop(0, (NCHUNK - 1) // 2, _p,
                          jnp.zeros((LANES,), _F32))
    sum16 = _consume(NCHUNK - 1, rows, sem, sum16)
    stage16[...] = sum16
    pltpu.sync_copy(stage16, sums_hbm.at[pl.ds(gtid * LANES, LANES)])
    plsc.subcore_barrier()

    for i in range(8):
        cid = s + i * NS
        @pl.when(cid < NN // CH)
        def _():
            pltpu.sync_copy(accsh.at[pl.ds(cid * CH, CH)], rows)
            pltpu.sync_copy(rows, out_hbm.at[c, pl.ds(cid * CH, CH)])


def _gat1_edges(h1, ssrc_arr, sdst_arr, src, dst, m1):
    mesh = plsc.VectorSubcoreMesh(core_axis_name="c", subcore_axis_name="s",
                                  num_cores=NC, num_subcores=NS)
    f = pl.kernel(
        _gat1_body,
        out_type=[
            jax.ShapeDtypeStruct((NC, NN, D1), _F32),
            jax.ShapeDtypeStruct((NC * NS * LANES,), _F32),
        ],
        mesh=mesh,
        compiler_params=pltpu.CompilerParams(needs_layout_passes=False,
                                             use_tc_tiling_on_sc=False),
        scratch_types=[
            pltpu.VMEM((NN,), _F32),
            pltpu.VMEM((NN,), _F32),
            pltpu.VMEM((1, LANES), _F32),
            pltpu.VMEM((EPT,), jnp.int32),
            pltpu.VMEM((EPT,), jnp.int32),
            pltpu.VMEM((CH,), _F32),
            pltpu.VMEM((LANES,), _F32),
            pltpu.VMEM((CH, D1), _F32),
            pltpu.VMEM((CH, D1), _F32),
            pltpu.VMEM_SHARED((NN, D1), _F32),
            pltpu.SemaphoreType.DMA,
            pltpu.SemaphoreType.DMA,
        ],
    )
    return f(h1, ssrc_arr, sdst_arr, src, dst, m1)


def _k3_body(parts_ref, sums_ref, w2_ref, a2_ref, h2_ref, s2_ref, m_ref):
    z1 = jnp.sum(sums_ref[...])
    v = parts_ref[...]
    p = v[0] + v[1]
    x1 = p * (1.0 / z1)
    x1 = jnp.where(x1 > 0.0, x1, jnp.exp(jnp.minimum(x1, 0.0)) - 1.0)
    h2t = lax.dot_general(w2_ref[...], x1, (((1,), (1,)), ((), ())),
                          preferred_element_type=_F32)
    h2_ref[...] = h2t
    a_src = a2_ref[0, 0]
    a_dst = a2_ref[0, 1]
    s2 = jnp.concatenate([a_src * h2t, a_dst * h2t], axis=0)
    s2_ref[...] = s2
    b = jnp.max(s2[0]) + jnp.max(s2[1])
    mb = jnp.where(b >= 0.0, b, 0.01 * b)
    m_ref[...] = jnp.broadcast_to(mb, (1, LANES))


def _mid(parts, sums1, w2, a2):
    return pl.pallas_call(
        _k3_body,
        out_shape=[
            jax.ShapeDtypeStruct((1, NN), _F32),
            jax.ShapeDtypeStruct((2, NN), _F32),
            jax.ShapeDtypeStruct((1, LANES), _F32),
        ],
    )(parts, sums1, w2, a2)


def _gat2_body(h_hbm, ssrc_hbm, sdst_hbm, src_hbm, dst_hbm, m_hbm,
               out_hbm, sums_hbm,
               ssrc, sdst, h2tab, mbuf, srcL, dstL, msg, stage16, zstage,
               acc2sh):
    c = lax.axis_index("c")
    s = lax.axis_index("s")
    gtid = c * NS + s

    pltpu.sync_copy(ssrc_hbm, ssrc)
    pltpu.sync_copy(sdst_hbm, sdst)
    pltpu.sync_copy(h_hbm, h2tab)
    pltpu.sync_copy(m_hbm, mbuf)
    e0 = gtid * EPT
    pltpu.sync_copy(src_hbm.at[pl.ds(e0, EPT)], srcL)
    pltpu.sync_copy(dst_hbm.at[pl.ds(e0, EPT)], dstL)
    mv = mbuf[0, pl.ds(0, LANES)]

    def _z(i, _):
        zstage[pl.ds(i * LANES, LANES)] = jnp.zeros((LANES,), _F32)
        return 0
    lax.fori_loop(0, 40, _z, 0)

    @pl.when(s < NS - 1)
    def _():
        pltpu.sync_copy(zstage.at[pl.ds(0, 640)],
                        acc2sh.at[pl.ds(s * 640, 640)])

    @pl.when(s == NS - 1)
    def _():
        pltpu.sync_copy(zstage.at[pl.ds(0, 400)], acc2sh.at[pl.ds(9600, 400)])
    plsc.subcore_barrier()

    def _p(q, s16):
        b = q * CH
        for g in range(CH // LANES):
            sl = pl.ds(b + g * LANES, LANES)
            i16 = srcL[sl]
            u = plsc.load_gather(ssrc, [i16]) \
                + plsc.load_gather(sdst, [dstL[sl]])
            e16 = jnp.where(u >= 0.0, u, 0.01 * u)
            w16 = jnp.exp(e16 - mv)
            msg[pl.ds(g * LANES, LANES)] = plsc.load_gather(h2tab, [i16]) * w16
            s16 = s16 + w16
        pltpu.sync_copy(msg, acc2sh.at[dstL.at[pl.ds(b, CH)]], add=True)
        return s16

    sum16 = lax.fori_loop(0, NCHUNK, _p, jnp.zeros((LANES,), _F32))
    stage16[...] = sum16
    pltpu.sync_copy(stage16, sums_hbm.at[pl.ds(gtid * LANES, LANES)])
    plsc.subcore_barrier()

    @pl.when(s < NS - 1)
    def _():
        pltpu.sync_copy(acc2sh.at[pl.ds(s * 640, 640)],
                        zstage.at[pl.ds(0, 640)])
        pltpu.sync_copy(zstage.at[pl.ds(0, 640)],
                        out_hbm.at[pl.ds(c * NN + s * 640, 640)])

    @pl.when(s == NS - 1)
    def _():
        pltpu.sync_copy(acc2sh.at[pl.ds(9600, 400)], zstage.at[pl.ds(0, 400)])
        pltpu.sync_copy(zstage.at[pl.ds(0, 400)],
                        out_hbm.at[pl.ds(c * NN + 9600, 400)])


def _gat2_edges(h2, ssrc_arr, sdst_arr, src, dst, m2):
    mesh = plsc.VectorSubcoreMesh(core_axis_name="c", subcore_axis_name="s",
                                  num_cores=NC, num_subcores=NS)
    f = pl.kernel(
        _gat2_body,
        out_type=[
            jax.ShapeDtypeStruct((NC * NN,), _F32),
            jax.ShapeDtypeStruct((NC * NS * LANES,), _F32),
        ],
        mesh=mesh,
        compiler_params=pltpu.CompilerParams(needs_layout_passes=False,
                                             use_tc_tiling_on_sc=False),
        scratch_types=[
            pltpu.VMEM((NN,), _F32),
            pltpu.VMEM((NN,), _F32),
            pltpu.VMEM((NN,), _F32),
            pltpu.VMEM((1, LANES), _F32),
            pltpu.VMEM((EPT,), jnp.int32),
            pltpu.VMEM((EPT,), jnp.int32),
            pltpu.VMEM((CH,), _F32),
            pltpu.VMEM((LANES,), _F32),
            pltpu.VMEM((640,), _F32),
            pltpu.VMEM_SHARED((NN,), _F32),
        ],
    )
    return f(h2, ssrc_arr, sdst_arr, src, dst, m2)


def _k5_body(parts_ref, sums_ref, ci_ref, cr_ref, n_ref, i_ref, r_ref,
             wih_ref, bih_ref, bhh_ref,
             wih_h_ref, wic_ref, bi_ref, wrh_ref, wrc_ref, br_ref,
             wsh_ref, wsc_ref, bs_ref,
             predI_ref, predR_ref, phyI_ref, phyR_ref, h_ref):
    z2 = jnp.sum(sums_ref[...])
    v = parts_ref[...]
    p = v[0:1, :] + v[1:2, :]
    x2 = p * (1.0 / z2)
    x2 = jnp.where(x2 > 0.0, x2, jnp.exp(jnp.minimum(x2, 0.0)) - 1.0)

    gi = wih_ref[...] * x2 + bih_ref[...]
    gh = bhh_ref[...]
    G = 32
    r = jax.nn.sigmoid(gi[0:G, :] + gh[0:G, :])
    z = jax.nn.sigmoid(gi[G:2 * G, :] + gh[G:2 * G, :])
    n = jnp.tanh(gi[2 * G:3 * G, :] + r * gh[2 * G:3 * G, :])
    h_new = (1.0 - z) * n
    h_ref[...] = h_new

    ci = ci_ref[...]
    cr = cr_ref[...]

    def head(wh_ref, wc_ref, b_ref):
        t = lax.dot_general(wh_ref[...], h_new, (((1,), (0,)), ((), ())),
                            preferred_element_type=_F32)
        return t + wc_ref[:, 0:1] * ci + wc_ref[:, 1:2] * cr + b_ref[...]

    predI_ref[...] = head(wih_h_ref, wic_ref, bi_ref)
    predR_ref[...] = head(wrh_ref, wrc_ref, br_ref)
    sir = jax.nn.sigmoid(head(wsh_ref, wsc_ref, bs_ref))
    alpha = sir[0:1, :]
    beta = sir[1:2, :]
    Nv = n_ref[...]
    Iv = i_ref[...]
    Rv = r_ref[...]
    Sv = jnp.maximum(Nv - Iv - Rv, 0.0)
    phyI_ref[...] = alpha * Iv * (Sv / Nv) - beta * Iv
    phyR_ref[...] = beta * Iv


def _final(parts2, sums2, ci, cr, nv, iv, rv, wih, bih, bhh,
           wi_h, wi_c, bi, wr_h, wr_c, br, ws_h, ws_c, bs):
    PW = 5
    G = 32
    return pl.pallas_call(
        _k5_body,
        out_shape=[
            jax.ShapeDtypeStruct((PW, NN), _F32),
            jax.ShapeDtypeStruct((PW, NN), _F32),
            jax.ShapeDtypeStruct((1, NN), _F32),
            jax.ShapeDtypeStruct((1, NN), _F32),
            jax.ShapeDtypeStruct((G, NN), _F32),
        ],
    )(parts2, sums2, ci, cr, nv, iv, rv, wih, bih, bhh,
      wi_h, wi_c, bi, wr_h, wr_c, br, ws_h, ws_c, bs)


def kernel(dynamic, cI, cR, N, I, R, edge_index, W1, a1, W2, a2,
           W_ih, W_hh, b_ih, b_hh, WI, bI, WR, bR, Ws, bs):
    x = dynamic.reshape(NN, IN_DIM)
    src = edge_index[0]
    dst = edge_index[1]

    h1, s1, m1 = _proj1(x, W1, a1.reshape(2, D1))
    parts1, sums1 = _gat1_edges(h1, s1[0], s1[1], src, dst, m1)

    h2row, s2, m2 = _mid(parts1, sums1, W2, a2)
    parts2, sums2 = _gat2_edges(h2row.reshape(NN), s2[0], s2[1], src, dst, m2)

    G = 32
    PW = 5
    pI, pR, fI, fR, hT = _final(
        parts2.reshape(NC, NN), sums2, cI, cR,
        N.reshape(1, NN), I.reshape(1, NN), R.reshape(1, NN),
        W_ih, b_ih.reshape(3 * G, 1), b_hh.reshape(3 * G, 1),
        WI[:, :G], WI[:, G:], bI.reshape(PW, 1),
        WR[:, :G], WR[:, G:], bR.reshape(PW, 1),
        Ws[:, :G], Ws[:, G:], bs.reshape(2, 1),
    )

    pred_I = pI.T.reshape(NN, 1, PW)
    pred_R = pR.T.reshape(NN, 1, PW)
    phy_I = fI.T.reshape(NN, 1, 1)
    phy_R = fR.T.reshape(NN, 1, 1)
    h_state = hT.T.reshape(NN, 1, G)
    return (pred_I, pred_R, phy_I, phy_R, h_state)


# speedup vs baseline: 33.1310x; 1.3192x over previous
import jax
import jax.numpy as jnp
from jax import lax
from jax.experimental import pallas as pl
from jax.experimental.pallas import tpu as pltpu
from jax.experimental.pallas import tpu_sc as plsc

NN = 10000
NE = 320000
IN_DIM = 128
D1 = 64
NC = 2
NS = 16
LANES = 16
EPT = NE // (NS * NC)
CH = 80
NCHUNK = EPT // CH

_F32 = jnp.float32


def _splat_from(ref, pos):
    idx = jnp.full((LANES,), pos, jnp.int32)
    return plsc.load_gather(ref, [idx])


def _k1_body(x_ref, w1_ref, a_ref, h_ref, s_ref, m_ref, bd_ref):
    xb = x_ref[...]
    h = lax.dot_general(xb, w1_ref[...], (((1,), (1,)), ((), ())),
                        preferred_element_type=_F32)
    s = lax.dot_general(a_ref[...], h, (((1,), (1,)), ((), ())),
                        preferred_element_type=_F32)
    s_ref[...] = s
    ms = jnp.max(s[0])
    md = jnp.max(s[1])
    b = ms + md
    mb = jnp.where(b >= 0.0, b, 0.01 * b)
    m_ref[...] = jnp.broadcast_to(mb, (1, LANES))
    av = jnp.exp(s[0:1] - ms)
    cv = jnp.exp(0.01 * (s[0:1] - ms))
    h_ref[0:NN] = h * av.T
    h_ref[NN:2 * NN] = h * cv.T
    bv = jnp.exp(s[1:2] - md) * jnp.exp(b - mb)
    dv = jnp.exp(0.01 * (s[1:2] - md)) * jnp.exp(0.01 * b - mb)
    bd_ref[...] = jnp.concatenate([bv, dv], axis=0)


def _proj1(x, w1, a2x64):
    return pl.pallas_call(
        _k1_body,
        out_shape=[
            jax.ShapeDtypeStruct((2 * NN, D1), _F32),
            jax.ShapeDtypeStruct((2, NN), _F32),
            jax.ShapeDtypeStruct((1, LANES), _F32),
            jax.ShapeDtypeStruct((2, NN), _F32),
        ],
    )(x, w1, a2x64)


def _gat1_body(h_hbm, ssrc_hbm, sdst_hbm, src_hbm, dst_hbm, m_hbm,
               out_hbm, sums_hbm,
               ssrc, sdst, mbuf, srcL, dstL, siA, siB, diA, diB,
               stage16, rows, rowsB, accsh, sem, semB):
    c = lax.axis_index("c")
    s = lax.axis_index("s")
    gtid = c * NS + s

    pltpu.sync_copy(ssrc_hbm, ssrc)
    pltpu.sync_copy(sdst_hbm, sdst)
    pltpu.sync_copy(m_hbm, mbuf)
    e0 = gtid * EPT
    pltpu.sync_copy(src_hbm.at[pl.ds(e0, EPT)], srcL)
    pltpu.sync_copy(dst_hbm.at[pl.ds(e0, EPT)], dstL)
    mv = mbuf[0, pl.ds(0, LANES)]

    def _zrow(i, _):
        for f in range(D1 // LANES):
            rows[i, pl.ds(f * LANES, LANES)] = jnp.zeros((LANES,), _F32)
        return 0
    lax.fori_loop(0, CH, _zrow, 0)
    for i in range(16):
        cid = s + i * NS
        @pl.when(cid < 2 * NN // CH)
        def _():
            pltpu.sync_copy(rows, accsh.at[pl.ds(cid * CH, CH)])
    plsc.subcore_barrier()

    def _cidx(q, si, di, s16):
        b = q * CH
        for g in range(CH // LANES):
            sl = pl.ds(b + g * LANES, LANES)
            s16i = srcL[sl]
            d16i = dstL[sl]
            u = plsc.load_gather(ssrc, [s16i]) \
                + plsc.load_gather(sdst, [d16i])
            e16 = jnp.where(u >= 0.0, u, 0.01 * u)
            s16 = s16 + jnp.exp(e16 - mv)
            off = jnp.where(u >= 0.0, jnp.int32(0), jnp.int32(NN))
            si[pl.ds(g * LANES, LANES)] = s16i + off
            di[pl.ds(g * LANES, LANES)] = d16i + off
        return s16

    def _issue(buf, si, psem):
        pltpu.async_copy(h_hbm.at[si], buf, psem)

    def _wait(buf, si, psem):
        pltpu.make_async_copy(h_hbm.at[si], buf, psem).wait()

    sum16 = _cidx(0, siA, diA, jnp.zeros((LANES,), _F32))
    _issue(rows, siA, sem)

    def _p(q2, s16):
        a = 2 * q2
        s16 = _cidx(a + 1, siB, diB, s16)
        _issue(rowsB, siB, semB)
        _wait(rows, siA, sem)
        pltpu.sync_copy(rows, accsh.at[diA], add=True)
        s16 = _cidx(a + 2, siA, diA, s16)
        _issue(rows, siA, sem)
        _wait(rowsB, siB, semB)
        pltpu.sync_copy(rowsB, accsh.at[diB], add=True)
        return s16

    sum16 = lax.fori_loop(0, (NCHUNK - 1) // 2, _p, sum16)
    _wait(rows, siA, sem)
    pltpu.sync_copy(rows, accsh.at[diA], add=True)

    stage16[...] = sum16
    pltpu.sync_copy(stage16, sums_hbm.at[pl.ds(gtid * LANES, LANES)])
    plsc.subcore_barrier()

    for i in range(16):
        cid = s + i * NS
        @pl.when(cid < 2 * NN // CH)
        def _():
            pltpu.sync_copy(accsh.at[pl.ds(cid * CH, CH)], rows)
            pltpu.sync_copy(rows, out_hbm.at[c, pl.ds(cid * CH, CH)])


def _gat1_edges(h1, ssrc_arr, sdst_arr, src, dst, m1):
    mesh = plsc.VectorSubcoreMesh(core_axis_name="c", subcore_axis_name="s",
                                  num_cores=NC, num_subcores=NS)
    f = pl.kernel(
        _gat1_body,
        out_type=[
            jax.ShapeDtypeStruct((NC, 2 * NN, D1), _F32),
            jax.ShapeDtypeStruct((NC * NS * LANES,), _F32),
        ],
        mesh=mesh,
        compiler_params=pltpu.CompilerParams(needs_layout_passes=False,
                                             use_tc_tiling_on_sc=False),
        scratch_types=[
            pltpu.VMEM((NN,), _F32),
            pltpu.VMEM((NN,), _F32),
            pltpu.VMEM((1, LANES), _F32),
            pltpu.VMEM((EPT,), jnp.int32),
            pltpu.VMEM((EPT,), jnp.int32),
            pltpu.VMEM((CH,), jnp.int32),
            pltpu.VMEM((CH,), jnp.int32),
            pltpu.VMEM((CH,), jnp.int32),
            pltpu.VMEM((CH,), jnp.int32),
            pltpu.VMEM((LANES,), _F32),
            pltpu.VMEM((CH, D1), _F32),
            pltpu.VMEM((CH, D1), _F32),
            pltpu.VMEM_SHARED((2 * NN, D1), _F32),
            pltpu.SemaphoreType.DMA,
            pltpu.SemaphoreType.DMA,
        ],
    )
    return f(h1, ssrc_arr, sdst_arr, src, dst, m1)


def _k3_body(parts_ref, sums_ref, bd_ref, w2_ref, a2_ref, h2_ref, s2_ref,
             m_ref):
    z1 = jnp.sum(sums_ref[...])
    v = parts_ref[...]
    bd = bd_ref[...]
    p = (v[0, 0:NN] + v[1, 0:NN]) * bd[0][:, None] \
        + (v[0, NN:2 * NN] + v[1, NN:2 * NN]) * bd[1][:, None]
    x1 = p * (1.0 / z1)
    x1 = jnp.where(x1 > 0.0, x1, jnp.exp(jnp.minimum(x1, 0.0)) - 1.0)
    h2t = lax.dot_general(w2_ref[...], x1, (((1,), (1,)), ((), ())),
                          preferred_element_type=_F32)
    h2_ref[...] = h2t
    a_src = a2_ref[0, 0]
    a_dst = a2_ref[0, 1]
    s2 = jnp.concatenate([a_src * h2t, a_dst * h2t], axis=0)
    s2_ref[...] = s2
    b = jnp.max(s2[0]) + jnp.max(s2[1])
    mb = jnp.where(b >= 0.0, b, 0.01 * b)
    m_ref[...] = jnp.broadcast_to(mb, (1, LANES))


def _mid(parts, sums1, bd, w2, a2):
    return pl.pallas_call(
        _k3_body,
        out_shape=[
            jax.ShapeDtypeStruct((1, NN), _F32),
            jax.ShapeDtypeStruct((2, NN), _F32),
            jax.ShapeDtypeStruct((1, LANES), _F32),
        ],
    )(parts, sums1, bd, w2, a2)


def _gat2_body(h_hbm, ssrc_hbm, sdst_hbm, src_hbm, dst_hbm, m_hbm,
               out_hbm, sums_hbm,
               ssrc, sdst, h2tab, mbuf, srcL, dstL, msg, stage16, zstage,
               acc2sh):
    c = lax.axis_index("c")
    s = lax.axis_index("s")
    gtid = c * NS + s

    pltpu.sync_copy(ssrc_hbm, ssrc)
    pltpu.sync_copy(sdst_hbm, sdst)
    pltpu.sync_copy(h_hbm, h2tab)
    pltpu.sync_copy(m_hbm, mbuf)
    e0 = gtid * EPT
    pltpu.sync_copy(src_hbm.at[pl.ds(e0, EPT)], srcL)
    pltpu.sync_copy(dst_hbm.at[pl.ds(e0, EPT)], dstL)
    mv = mbuf[0, pl.ds(0, LANES)]

    def _z(i, _):
        zstage[pl.ds(i * LANES, LANES)] = jnp.zeros((LANES,), _F32)
        return 0
    lax.fori_loop(0, 40, _z, 0)

    @pl.when(s < NS - 1)
    def _():
        pltpu.sync_copy(zstage.at[pl.ds(0, 640)],
                        acc2sh.at[pl.ds(s * 640, 640)])

    @pl.when(s == NS - 1)
    def _():
        pltpu.sync_copy(zstage.at[pl.ds(0, 400)], acc2sh.at[pl.ds(9600, 400)])
    plsc.subcore_barrier()

    def _p(q, s16):
        b = q * CH
        for g in range(CH // LANES):
            sl = pl.ds(b + g * LANES, LANES)
            i16 = srcL[sl]
            u = plsc.load_gather(ssrc, [i16]) \
                + plsc.load_gather(sdst, [dstL[sl]])
            e16 = jnp.where(u >= 0.0, u, 0.01 * u)
            w16 = jnp.exp(e16 - mv)
            msg[pl.ds(g * LANES, LANES)] = plsc.load_gather(h2tab, [i16]) * w16
            s16 = s16 + w16
        pltpu.sync_copy(msg, acc2sh.at[dstL.at[pl.ds(b, CH)]], add=True)
        return s16

    sum16 = lax.fori_loop(0, NCHUNK, _p, jnp.zeros((LANES,), _F32))
    stage16[...] = sum16
    pltpu.sync_copy(stage16, sums_hbm.at[pl.ds(gtid * LANES, LANES)])
    plsc.subcore_barrier()

    @pl.when(s < NS - 1)
    def _():
        pltpu.sync_copy(acc2sh.at[pl.ds(s * 640, 640)],
                        zstage.at[pl.ds(0, 640)])
        pltpu.sync_copy(zstage.at[pl.ds(0, 640)],
                        out_hbm.at[pl.ds(c * NN + s * 640, 640)])

    @pl.when(s == NS - 1)
    def _():
        pltpu.sync_copy(acc2sh.at[pl.ds(9600, 400)], zstage.at[pl.ds(0, 400)])
        pltpu.sync_copy(zstage.at[pl.ds(0, 400)],
                        out_hbm.at[pl.ds(c * NN + 9600, 400)])


def _gat2_edges(h2, ssrc_arr, sdst_arr, src, dst, m2):
    mesh = plsc.VectorSubcoreMesh(core_axis_name="c", subcore_axis_name="s",
                                  num_cores=NC, num_subcores=NS)
    f = pl.kernel(
        _gat2_body,
        out_type=[
            jax.ShapeDtypeStruct((NC * NN,), _F32),
            jax.ShapeDtypeStruct((NC * NS * LANES,), _F32),
        ],
        mesh=mesh,
        compiler_params=pltpu.CompilerParams(needs_layout_passes=False,
                                             use_tc_tiling_on_sc=False),
        scratch_types=[
            pltpu.VMEM((NN,), _F32),
            pltpu.VMEM((NN,), _F32),
            pltpu.VMEM((NN,), _F32),
            pltpu.VMEM((1, LANES), _F32),
            pltpu.VMEM((EPT,), jnp.int32),
            pltpu.VMEM((EPT,), jnp.int32),
            pltpu.VMEM((CH,), _F32),
            pltpu.VMEM((LANES,), _F32),
            pltpu.VMEM((640,), _F32),
            pltpu.VMEM_SHARED((NN,), _F32),
        ],
    )
    return f(h2, ssrc_arr, sdst_arr, src, dst, m2)


def _k5_body(parts_ref, sums_ref, ci_ref, cr_ref, n_ref, i_ref, r_ref,
             wih_ref, bih_ref, bhh_ref,
             wih_h_ref, wic_ref, bi_ref, wrh_ref, wrc_ref, br_ref,
             wsh_ref, wsc_ref, bs_ref,
             predI_ref, predR_ref, phyI_ref, phyR_ref, h_ref):
    z2 = jnp.sum(sums_ref[...])
    v = parts_ref[...]
    p = v[0:1, :] + v[1:2, :]
    x2 = p * (1.0 / z2)
    x2 = jnp.where(x2 > 0.0, x2, jnp.exp(jnp.minimum(x2, 0.0)) - 1.0)

    gi = wih_ref[...] * x2 + bih_ref[...]
    gh = bhh_ref[...]
    G = 32
    r = jax.nn.sigmoid(gi[0:G, :] + gh[0:G, :])
    z = jax.nn.sigmoid(gi[G:2 * G, :] + gh[G:2 * G, :])
    n = jnp.tanh(gi[2 * G:3 * G, :] + r * gh[2 * G:3 * G, :])
    h_new = (1.0 - z) * n
    h_ref[...] = h_new

    ci = ci_ref[...]
    cr = cr_ref[...]

    def head(wh_ref, wc_ref, b_ref):
        t = lax.dot_general(wh_ref[...], h_new, (((1,), (0,)), ((), ())),
                            preferred_element_type=_F32)
        return t + wc_ref[:, 0:1] * ci + wc_ref[:, 1:2] * cr + b_ref[...]

    predI_ref[...] = head(wih_h_ref, wic_ref, bi_ref)
    predR_ref[...] = head(wrh_ref, wrc_ref, br_ref)
    sir = jax.nn.sigmoid(head(wsh_ref, wsc_ref, bs_ref))
    alpha = sir[0:1, :]
    beta = sir[1:2, :]
    Nv = n_ref[...]
    Iv = i_ref[...]
    Rv = r_ref[...]
    Sv = jnp.maximum(Nv - Iv - Rv, 0.0)
    phyI_ref[...] = alpha * Iv * (Sv / Nv) - beta * Iv
    phyR_ref[...] = beta * Iv


def _final(parts2, sums2, ci, cr, nv, iv, rv, wih, bih, bhh,
           wi_h, wi_c, bi, wr_h, wr_c, br, ws_h, ws_c, bs):
    PW = 5
    G = 32
    return pl.pallas_call(
        _k5_body,
        out_shape=[
            jax.ShapeDtypeStruct((PW, NN), _F32),
            jax.ShapeDtypeStruct((PW, NN), _F32),
            jax.ShapeDtypeStruct((1, NN), _F32),
            jax.ShapeDtypeStruct((1, NN), _F32),
            jax.ShapeDtypeStruct((G, NN), _F32),
        ],
    )(parts2, sums2, ci, cr, nv, iv, rv, wih, bih, bhh,
      wi_h, wi_c, bi, wr_h, wr_c, br, ws_h, ws_c, bs)


def kernel(dynamic, cI, cR, N, I, R, edge_index, W1, a1, W2, a2,
           W_ih, W_hh, b_ih, b_hh, WI, bI, WR, bR, Ws, bs):
    x = dynamic.reshape(NN, IN_DIM)
    src = edge_index[0]
    dst = edge_index[1]

    hstack, s1, m1, bd1 = _proj1(x, W1, a1.reshape(2, D1))
    parts1, sums1 = _gat1_edges(hstack, s1[0], s1[1], src, dst, m1)

    h2row, s2, m2 = _mid(parts1, sums1, bd1, W2, a2)
    parts2, sums2 = _gat2_edges(h2row.reshape(NN), s2[0], s2[1], src, dst, m2)

    G = 32
    PW = 5
    pI, pR, fI, fR, hT = _final(
        parts2.reshape(NC, NN), sums2, cI, cR,
        N.reshape(1, NN), I.reshape(1, NN), R.reshape(1, NN),
        W_ih, b_ih.reshape(3 * G, 1), b_hh.reshape(3 * G, 1),
        WI[:, :G], WI[:, G:], bI.reshape(PW, 1),
        WR[:, :G], WR[:, G:], bR.reshape(PW, 1),
        Ws[:, :G], Ws[:, G:], bs.reshape(2, 1),
    )

    pred_I = pI.T.reshape(NN, 1, PW)
    pred_R = pR.T.reshape(NN, 1, PW)
    phy_I = fI.T.reshape(NN, 1, 1)
    phy_R = fR.T.reshape(NN, 1, 1)
    h_state = hT.T.reshape(NN, 1, G)
    return (pred_I, pred_R, phy_I, phy_R, h_state)


# speedup vs baseline: 35.4996x; 1.0715x over previous
import jax
import jax.numpy as jnp
from jax import lax
from jax.experimental import pallas as pl
from jax.experimental.pallas import tpu as pltpu
from jax.experimental.pallas import tpu_sc as plsc

NN = 10000
NE = 320000
IN_DIM = 128
D1 = 64
NC = 2
NS = 16
LANES = 16
EPT = NE // (NS * NC)
EPT1 = NE // NS
D1H = D1 // 2
CH = 80
NCHUNK = EPT // CH
NCHUNK1 = EPT1 // CH

_F32 = jnp.float32


def _splat_from(ref, pos):
    idx = jnp.full((LANES,), pos, jnp.int32)
    return plsc.load_gather(ref, [idx])


def _k1_body(x_ref, w1_ref, a_ref, h_ref, s_ref, m_ref, bd_ref):
    xb = x_ref[...]
    h = lax.dot_general(xb, w1_ref[...], (((1,), (1,)), ((), ())),
                        preferred_element_type=_F32)
    s = lax.dot_general(a_ref[...], h, (((1,), (1,)), ((), ())),
                        preferred_element_type=_F32)
    s_ref[...] = s
    ms = jnp.max(s[0])
    md = jnp.max(s[1])
    b = ms + md
    mb = jnp.where(b >= 0.0, b, 0.01 * b)
    m_ref[...] = jnp.broadcast_to(mb, (1, LANES))
    av = jnp.exp(s[0:1] - ms)
    cv = jnp.exp(0.01 * (s[0:1] - ms))
    ha = h * av.T
    hc = h * cv.T
    h_ref[0:NN] = ha[:, 0:D1H]
    h_ref[NN:2 * NN] = hc[:, 0:D1H]
    h_ref[2 * NN:3 * NN] = ha[:, D1H:D1]
    h_ref[3 * NN:4 * NN] = hc[:, D1H:D1]
    bv = jnp.exp(s[1:2] - md) * jnp.exp(b - mb)
    dv = jnp.exp(0.01 * (s[1:2] - md)) * jnp.exp(0.01 * b - mb)
    bd_ref[...] = jnp.concatenate([bv, dv], axis=0)


def _proj1(x, w1, a2x64):
    return pl.pallas_call(
        _k1_body,
        out_shape=[
            jax.ShapeDtypeStruct((4 * NN, D1H), _F32),
            jax.ShapeDtypeStruct((2, NN), _F32),
            jax.ShapeDtypeStruct((1, LANES), _F32),
            jax.ShapeDtypeStruct((2, NN), _F32),
        ],
    )(x, w1, a2x64)


def _gat1_body(h_hbm, ssrc_hbm, sdst_hbm, src_hbm, dst_hbm, m_hbm,
               out_hbm, sums_hbm,
               ssrc, sdst, mbuf, srcL, dstL, stage16,
               r0, r1, r2, r3, r4, r5, r6, r7,
               i0, i1, i2, i3, i4, i5, i6, i7,
               d0, d1, d2, d3, d4, d5, d6, d7,
               accsh,
               g0, g1, g2, g3, g4, g5, g6, g7,
               s0, s1, s2, s3, s4, s5, s6, s7):
    rows = [r0, r1, r2, r3, r4, r5, r6, r7]
    si = [i0, i1, i2, i3, i4, i5, i6, i7]
    di = [d0, d1, d2, d3, d4, d5, d6, d7]
    gsem = [g0, g1, g2, g3, g4, g5, g6, g7]
    ssem = [s0, s1, s2, s3, s4, s5, s6, s7]
    c = lax.axis_index("c")
    s = lax.axis_index("s")
    gtid = c * NS + s

    pltpu.sync_copy(ssrc_hbm, ssrc)
    pltpu.sync_copy(sdst_hbm, sdst)
    pltpu.sync_copy(m_hbm, mbuf)
    e0 = s * EPT1
    pltpu.sync_copy(src_hbm.at[pl.ds(e0, EPT1)], srcL)
    pltpu.sync_copy(dst_hbm.at[pl.ds(e0, EPT1)], dstL)
    mv = mbuf[0, pl.ds(0, LANES)]
    c2 = c * (2 * NN)

    def _zrow(i, _):
        for f in range(D1H // LANES):
            r0[i, pl.ds(f * LANES, LANES)] = jnp.zeros((LANES,), _F32)
        return 0
    lax.fori_loop(0, CH, _zrow, 0)
    for i in range(16):
        cid = s + i * NS
        @pl.when(cid < 2 * NN // CH)
        def _():
            pltpu.sync_copy(r0, accsh.at[pl.ds(cid * CH, CH)])
    plsc.subcore_barrier()

    K = 8
    AHEAD = 4

    def _cidx(q, si, di, s16):
        b = q * CH
        for g in range(CH // LANES):
            sl = pl.ds(b + g * LANES, LANES)
            s16i = srcL[sl]
            d16i = dstL[sl]
            u = plsc.load_gather(ssrc, [s16i]) \
                + plsc.load_gather(sdst, [d16i])
            e16 = jnp.where(u >= 0.0, u, 0.01 * u)
            s16 = s16 + jnp.exp(e16 - mv)
            off = jnp.where(u >= 0.0, jnp.int32(0), jnp.int32(NN))
            si[pl.ds(g * LANES, LANES)] = s16i + off + c2
            di[pl.ds(g * LANES, LANES)] = d16i + off
        return s16

    def _prep(qp, bp, s16, first):
        if not first:
            pred = qp >= K
            if isinstance(pred, bool):
                if pred:
                    pltpu.make_async_copy(rows[bp], accsh.at[di[bp]],
                                          ssem[bp]).wait()
            else:
                @pl.when(pred)
                def _():
                    pltpu.make_async_copy(rows[bp], accsh.at[di[bp]],
                                          ssem[bp]).wait()
        s16 = _cidx(qp, si[bp], di[bp], s16)
        pltpu.async_copy(h_hbm.at[si[bp]], rows[bp], gsem[bp])
        return s16

    def _proc(b):
        pltpu.make_async_copy(h_hbm.at[si[b]], rows[b], gsem[b]).wait()
        pltpu.async_copy(rows[b], accsh.at[di[b]], ssem[b], add=True)

    sum16 = jnp.zeros((LANES,), _F32)
    for q in range(AHEAD):
        sum16 = _prep(q, q % K, sum16, True)

    def _step(i, s16):
        q0 = i * K
        for j in range(K):
            s16 = _prep(q0 + j + AHEAD, (j + AHEAD) % K, s16, False)
            _proc(j)
        return s16

    sum16 = lax.fori_loop(0, (NCHUNK1 - AHEAD - 1) // K, _step, sum16)
    for q in range(NCHUNK1 - AHEAD - 1 - (NCHUNK1 - AHEAD - 1) % K, NCHUNK1):
        b = q % K
        qp = q + AHEAD
        if qp < NCHUNK1:
            sum16 = _prep(qp, qp % K, sum16, False)
        _proc(b)
    for q in range(NCHUNK1 - K, NCHUNK1):
        b = q % K
        pltpu.make_async_copy(rows[b], accsh.at[di[b]], ssem[b]).wait()

    stage16[...] = sum16
    pltpu.sync_copy(stage16, sums_hbm.at[pl.ds(gtid * LANES, LANES)])
    plsc.subcore_barrier()

    for i in range(16):
        cid = s + i * NS
        @pl.when(cid < 2 * NN // CH)
        def _():
            pltpu.sync_copy(accsh.at[pl.ds(cid * CH, CH)], r0)
            pltpu.sync_copy(r0, out_hbm.at[c, pl.ds(cid * CH, CH)])


def _gat1_edges(h1, ssrc_arr, sdst_arr, src, dst, m1):
    mesh = plsc.VectorSubcoreMesh(core_axis_name="c", subcore_axis_name="s",
                                  num_cores=NC, num_subcores=NS)
    f = pl.kernel(
        _gat1_body,
        out_type=[
            jax.ShapeDtypeStruct((NC, 2 * NN, D1H), _F32),
            jax.ShapeDtypeStruct((NC * NS * LANES,), _F32),
        ],
        mesh=mesh,
        compiler_params=pltpu.CompilerParams(needs_layout_passes=False,
                                             use_tc_tiling_on_sc=False),
        scratch_types=(
            [
                pltpu.VMEM((NN,), _F32),
                pltpu.VMEM((NN,), _F32),
                pltpu.VMEM((1, LANES), _F32),
                pltpu.VMEM((EPT1,), jnp.int32),
                pltpu.VMEM((EPT1,), jnp.int32),
                pltpu.VMEM((LANES,), _F32),
            ]
            + [pltpu.VMEM((CH, D1H), _F32)] * 8
            + [pltpu.VMEM((CH,), jnp.int32)] * 8
            + [pltpu.VMEM((CH,), jnp.int32)] * 8
            + [pltpu.VMEM_SHARED((2 * NN, D1H), _F32)]
            + [pltpu.SemaphoreType.DMA] * 16
        ),
    )
    return f(h1, ssrc_arr, sdst_arr, src, dst, m1)


def _k3_body(parts_ref, sums_ref, bd_ref, w2_ref, a2_ref, h2_ref, s2_ref,
             m_ref):
    z1 = jnp.sum(sums_ref[...]) * 0.5
    v = parts_ref[...]
    bd = bd_ref[...]
    pos = jnp.concatenate([v[0, 0:NN], v[1, 0:NN]], axis=1)
    neg = jnp.concatenate([v[0, NN:2 * NN], v[1, NN:2 * NN]], axis=1)
    p = pos * bd[0][:, None] + neg * bd[1][:, None]
    x1 = p * (1.0 / z1)
    x1 = jnp.where(x1 > 0.0, x1, jnp.exp(jnp.minimum(x1, 0.0)) - 1.0)
    h2t = lax.dot_general(w2_ref[...], x1, (((1,), (1,)), ((), ())),
                          preferred_element_type=_F32)
    h2_ref[...] = h2t
    a_src = a2_ref[0, 0]
    a_dst = a2_ref[0, 1]
    s2 = jnp.concatenate([a_src * h2t, a_dst * h2t], axis=0)
    s2_ref[...] = s2
    b = jnp.max(s2[0]) + jnp.max(s2[1])
    mb = jnp.where(b >= 0.0, b, 0.01 * b)
    m_ref[...] = jnp.broadcast_to(mb, (1, LANES))


def _mid(parts, sums1, bd, w2, a2):
    return pl.pallas_call(
        _k3_body,
        out_shape=[
            jax.ShapeDtypeStruct((1, NN), _F32),
            jax.ShapeDtypeStruct((2, NN), _F32),
            jax.ShapeDtypeStruct((1, LANES), _F32),
        ],
    )(parts, sums1, bd, w2, a2)


def _gat2_body(h_hbm, ssrc_hbm, sdst_hbm, src_hbm, dst_hbm, m_hbm,
               out_hbm, sums_hbm,
               ssrc, sdst, h2tab, mbuf, srcL, dstL, msg, stage16, zstage,
               acc2sh):
    c = lax.axis_index("c")
    s = lax.axis_index("s")
    gtid = c * NS + s

    pltpu.sync_copy(ssrc_hbm, ssrc)
    pltpu.sync_copy(sdst_hbm, sdst)
    pltpu.sync_copy(h_hbm, h2tab)
    pltpu.sync_copy(m_hbm, mbuf)
    e0 = gtid * EPT
    pltpu.sync_copy(src_hbm.at[pl.ds(e0, EPT)], srcL)
    pltpu.sync_copy(dst_hbm.at[pl.ds(e0, EPT)], dstL)
    mv = mbuf[0, pl.ds(0, LANES)]

    def _z(i, _):
        zstage[pl.ds(i * LANES, LANES)] = jnp.zeros((LANES,), _F32)
        return 0
    lax.fori_loop(0, 40, _z, 0)

    @pl.when(s < NS - 1)
    def _():
        pltpu.sync_copy(zstage.at[pl.ds(0, 640)],
                        acc2sh.at[pl.ds(s * 640, 640)])

    @pl.when(s == NS - 1)
    def _():
        pltpu.sync_copy(zstage.at[pl.ds(0, 400)], acc2sh.at[pl.ds(9600, 400)])
    plsc.subcore_barrier()

    def _p(q, s16):
        b = q * CH
        for g in range(CH // LANES):
            sl = pl.ds(b + g * LANES, LANES)
            i16 = srcL[sl]
            u = plsc.load_gather(ssrc, [i16]) \
                + plsc.load_gather(sdst, [dstL[sl]])
            e16 = jnp.where(u >= 0.0, u, 0.01 * u)
            w16 = jnp.exp(e16 - mv)
            msg[pl.ds(g * LANES, LANES)] = plsc.load_gather(h2tab, [i16]) * w16
            s16 = s16 + w16
        pltpu.sync_copy(msg, acc2sh.at[dstL.at[pl.ds(b, CH)]], add=True)
        return s16

    sum16 = lax.fori_loop(0, NCHUNK, _p, jnp.zeros((LANES,), _F32))
    stage16[...] = sum16
    pltpu.sync_copy(stage16, sums_hbm.at[pl.ds(gtid * LANES, LANES)])
    plsc.subcore_barrier()

    @pl.when(s < NS - 1)
    def _():
        pltpu.sync_copy(acc2sh.at[pl.ds(s * 640, 640)],
                        zstage.at[pl.ds(0, 640)])
        pltpu.sync_copy(zstage.at[pl.ds(0, 640)],
                        out_hbm.at[pl.ds(c * NN + s * 640, 640)])

    @pl.when(s == NS - 1)
    def _():
        pltpu.sync_copy(acc2sh.at[pl.ds(9600, 400)], zstage.at[pl.ds(0, 400)])
        pltpu.sync_copy(zstage.at[pl.ds(0, 400)],
                        out_hbm.at[pl.ds(c * NN + 9600, 400)])


def _gat2_edges(h2, ssrc_arr, sdst_arr, src, dst, m2):
    mesh = plsc.VectorSubcoreMesh(core_axis_name="c", subcore_axis_name="s",
                                  num_cores=NC, num_subcores=NS)
    f = pl.kernel(
        _gat2_body,
        out_type=[
            jax.ShapeDtypeStruct((NC * NN,), _F32),
            jax.ShapeDtypeStruct((NC * NS * LANES,), _F32),
        ],
        mesh=mesh,
        compiler_params=pltpu.CompilerParams(needs_layout_passes=False,
                                             use_tc_tiling_on_sc=False),
        scratch_types=[
            pltpu.VMEM((NN,), _F32),
            pltpu.VMEM((NN,), _F32),
            pltpu.VMEM((NN,), _F32),
            pltpu.VMEM((1, LANES), _F32),
            pltpu.VMEM((EPT,), jnp.int32),
            pltpu.VMEM((EPT,), jnp.int32),
            pltpu.VMEM((CH,), _F32),
            pltpu.VMEM((LANES,), _F32),
            pltpu.VMEM((640,), _F32),
            pltpu.VMEM_SHARED((NN,), _F32),
        ],
    )
    return f(h2, ssrc_arr, sdst_arr, src, dst, m2)


def _k5_body(parts_ref, sums_ref, ci_ref, cr_ref, n_ref, i_ref, r_ref,
             wih_ref, bih_ref, bhh_ref,
             wih_h_ref, wic_ref, bi_ref, wrh_ref, wrc_ref, br_ref,
             wsh_ref, wsc_ref, bs_ref,
             predI_ref, predR_ref, phyI_ref, phyR_ref, h_ref):
    z2 = jnp.sum(sums_ref[...])
    v = parts_ref[...]
    p = v[0:1, :] + v[1:2, :]
    x2 = p * (1.0 / z2)
    x2 = jnp.where(x2 > 0.0, x2, jnp.exp(jnp.minimum(x2, 0.0)) - 1.0)

    gi = wih_ref[...] * x2 + bih_ref[...]
    gh = bhh_ref[...]
    G = 32
    r = jax.nn.sigmoid(gi[0:G, :] + gh[0:G, :])
    z = jax.nn.sigmoid(gi[G:2 * G, :] + gh[G:2 * G, :])
    n = jnp.tanh(gi[2 * G:3 * G, :] + r * gh[2 * G:3 * G, :])
    h_new = (1.0 - z) * n
    h_ref[...] = h_new

    ci = ci_ref[...]
    cr = cr_ref[...]

    def head(wh_ref, wc_ref, b_ref):
        t = lax.dot_general(wh_ref[...], h_new, (((1,), (0,)), ((), ())),
                            preferred_element_type=_F32)
        return t + wc_ref[:, 0:1] * ci + wc_ref[:, 1:2] * cr + b_ref[...]

    predI_ref[...] = head(wih_h_ref, wic_ref, bi_ref)
    predR_ref[...] = head(wrh_ref, wrc_ref, br_ref)
    sir = jax.nn.sigmoid(head(wsh_ref, wsc_ref, bs_ref))
    alpha = sir[0:1, :]
    beta = sir[1:2, :]
    Nv = n_ref[...]
    Iv = i_ref[...]
    Rv = r_ref[...]
    Sv = jnp.maximum(Nv - Iv - Rv, 0.0)
    phyI_ref[...] = alpha * Iv * (Sv / Nv) - beta * Iv
    phyR_ref[...] = beta * Iv


def _final(parts2, sums2, ci, cr, nv, iv, rv, wih, bih, bhh,
           wi_h, wi_c, bi, wr_h, wr_c, br, ws_h, ws_c, bs):
    PW = 5
    G = 32
    return pl.pallas_call(
        _k5_body,
        out_shape=[
            jax.ShapeDtypeStruct((PW, NN), _F32),
            jax.ShapeDtypeStruct((PW, NN), _F32),
            jax.ShapeDtypeStruct((1, NN), _F32),
            jax.ShapeDtypeStruct((1, NN), _F32),
            jax.ShapeDtypeStruct((G, NN), _F32),
        ],
    )(parts2, sums2, ci, cr, nv, iv, rv, wih, bih, bhh,
      wi_h, wi_c, bi, wr_h, wr_c, br, ws_h, ws_c, bs)


def kernel(dynamic, cI, cR, N, I, R, edge_index, W1, a1, W2, a2,
           W_ih, W_hh, b_ih, b_hh, WI, bI, WR, bR, Ws, bs):
    x = dynamic.reshape(NN, IN_DIM)
    src = edge_index[0]
    dst = edge_index[1]

    hstack, s1, m1, bd1 = _proj1(x, W1, a1.reshape(2, D1))
    parts1, sums1 = _gat1_edges(hstack, s1[0], s1[1], src, dst, m1)

    h2row, s2, m2 = _mid(parts1, sums1, bd1, W2, a2)
    parts2, sums2 = _gat2_edges(h2row.reshape(NN), s2[0], s2[1], src, dst, m2)

    G = 32
    PW = 5
    pI, pR, fI, fR, hT = _final(
        parts2.reshape(NC, NN), sums2, cI, cR,
        N.reshape(1, NN), I.reshape(1, NN), R.reshape(1, NN),
        W_ih, b_ih.reshape(3 * G, 1), b_hh.reshape(3 * G, 1),
        WI[:, :G], WI[:, G:], bI.reshape(PW, 1),
        WR[:, :G], WR[:, G:], bR.reshape(PW, 1),
        Ws[:, :G], Ws[:, G:], bs.reshape(2, 1),
    )

    pred_I = pI.T.reshape(NN, 1, PW)
    pred_R = pR.T.reshape(NN, 1, PW)
    phy_I = fI.T.reshape(NN, 1, 1)
    phy_R = fR.T.reshape(NN, 1, 1)
    h_state = hT.T.reshape(NN, 1, G)
    return (pred_I, pred_R, phy_I, phy_R, h_state)
